# Initial kernel scaffold; baseline (speedup 1.0000x reference)
import jax
import jax.numpy as jnp
from jax.experimental import pallas as pl


def kernel(tokens, table):
    raise NotImplementedError("write your pallas kernel here")


# speedup vs baseline: 3.0380x; 3.0380x over previous
import functools

import jax
import jax.numpy as jnp
import numpy as np
from jax import lax
from jax.experimental import pallas as pl
from jax.experimental.pallas import tpu as pltpu
from jax.experimental.pallas import tpu_sc as plsc

VOCAB = 100000
MAX_LEN = 200
D_MODEL = 64
BATCH = 4096

ROWS = BATCH * MAX_LEN
NC, NS = 2, 16
NW = NC * NS
RPW = ROWS // NW
G = 100
NG = RPW // G
NBUF = 4
NOUT = NG // NBUF
NVPR = D_MODEL // 16


def _positional_encoding() -> np.ndarray:
    pos = np.arange(MAX_LEN, dtype=np.float32)[:, None]
    i = np.arange(D_MODEL // 2, dtype=np.float32)[None, :]
    denom = np.power(10000.0, (2.0 * i) / D_MODEL)
    pe = np.zeros((MAX_LEN, D_MODEL), dtype=np.float32)
    pe[:, 0::2] = np.sin(pos / denom)
    pe[:, 1::2] = np.cos(pos / denom)
    return pe


@functools.partial(
    pl.kernel,
    out_type=jax.ShapeDtypeStruct((ROWS * D_MODEL,), jnp.float32),
    mesh=plsc.VectorSubcoreMesh(core_axis_name="c", subcore_axis_name="s"),
    scratch_types=[
        pltpu.VMEM((NG, G), jnp.int32),
        pltpu.VMEM((MAX_LEN, D_MODEL), jnp.float32),
        pltpu.VMEM((NBUF, G, D_MODEL), jnp.float32),
        pltpu.VMEM((NBUF, G * D_MODEL), jnp.float32),
        pltpu.SemaphoreType.DMA((NBUF,)),
        pltpu.SemaphoreType.DMA((NBUF,)),
    ],
    compiler_params=pltpu.CompilerParams(use_tc_tiling_on_sc=False),
)
def _sc_embed(tok_hbm, table_hbm, pe_hbm, out_hbm,
              idx_v, pe_v, gbuf, sbuf, gsem, ssem):
    wid = lax.axis_index("s") * NC + lax.axis_index("c")

    pltpu.sync_copy(tok_hbm.at[wid], idx_v)
    pltpu.sync_copy(pe_hbm, pe_v)

    for b in range(NBUF):
        pltpu.async_copy(table_hbm.at[idx_v.at[b]], gbuf.at[b], gsem.at[b])

    @pl.loop(0, NOUT)
    def _outer(o):
        for b in range(NBUF):
            g = o * NBUF + b
            pltpu.make_async_copy(
                table_hbm.at[idx_v.at[g]], gbuf.at[b], gsem.at[b]).wait()
            @pl.when(o > 0)
            def _():
                pltpu.make_async_copy(
                    sbuf.at[b],
                    out_hbm.at[pl.ds(0, G * D_MODEL)],
                    ssem.at[b]).wait()

            lbase = (b % 2) * G

            @pl.loop(0, G, unroll=4)
            def _row(r):
                for j in range(NVPR):
                    sl = pl.ds(j * 16, 16)
                    sbuf[b, pl.ds((r * NVPR + j) * 16, 16)] = (
                        gbuf[b, r, sl] + pe_v[lbase + r, sl])

            @pl.when(o < NOUT - 1)
            def _():
                pltpu.async_copy(table_hbm.at[idx_v.at[g + NBUF]],
                                 gbuf.at[b], gsem.at[b])

            pltpu.async_copy(
                sbuf.at[b],
                out_hbm.at[pl.ds((wid * RPW + g * G) * D_MODEL, G * D_MODEL)],
                ssem.at[b])

    for b in range(NBUF):
        pltpu.make_async_copy(
            sbuf.at[b], out_hbm.at[pl.ds(0, G * D_MODEL)], ssem.at[b]).wait()


def kernel(tokens, table):
    pe = jnp.asarray(_positional_encoding())
    tok3 = tokens.reshape(NW, NG, G)
    out = _sc_embed(tok3, table, pe)
    return out.reshape(BATCH, MAX_LEN, D_MODEL)


# speedup vs baseline: 3.3908x; 1.1161x over previous
import functools

import jax
import jax.numpy as jnp
import numpy as np
from jax import lax
from jax.experimental import pallas as pl
from jax.experimental.pallas import tpu as pltpu
from jax.experimental.pallas import tpu_sc as plsc

VOCAB = 100000
MAX_LEN = 200
D_MODEL = 64
BATCH = 4096

NC, NS = 2, 16
NW = NC * NS
BPW = BATCH // NW
G = MAX_LEN // 2
NBUF = 4
SNB = 2
NVPR = D_MODEL // 16


@functools.partial(
    pl.kernel,
    out_type=jax.ShapeDtypeStruct((BATCH, MAX_LEN, D_MODEL), jnp.float32),
    mesh=plsc.VectorSubcoreMesh(core_axis_name="c", subcore_axis_name="s"),
    scratch_types=[
        pltpu.VMEM((2 * BPW, G), jnp.int32),
        pltpu.VMEM((MAX_LEN, D_MODEL), jnp.float32),
        pltpu.VMEM((NBUF, MAX_LEN, D_MODEL), jnp.float32),
        pltpu.VMEM((SNB, MAX_LEN, D_MODEL), jnp.float32),
        pltpu.SemaphoreType.DMA((NBUF,)),
        pltpu.SemaphoreType.DMA((SNB,)),
    ],
    compiler_params=pltpu.CompilerParams(use_tc_tiling_on_sc=False),
)
def _sc_embed(tok_hbm, table_hbm, pe_hbm, out_hbm,
              idx_v, pe_v, gbuf, sbuf, gsem, ssem):
    wid = lax.axis_index("s") * NC + lax.axis_index("c")

    pltpu.sync_copy(tok_hbm.at[wid], idx_v)
    pltpu.sync_copy(pe_hbm, pe_v)

    def fire_gather(i, b):
        pltpu.async_copy(table_hbm.at[idx_v.at[2 * i]],
                         gbuf.at[b, pl.ds(0, G)], gsem.at[b])
        pltpu.async_copy(table_hbm.at[idx_v.at[2 * i + 1]],
                         gbuf.at[b, pl.ds(G, G)], gsem.at[b])

    def wait_gather(b):
        for _ in range(2):
            pltpu.make_async_copy(
                table_hbm.at[idx_v.at[0]],
                gbuf.at[b, pl.ds(0, G)], gsem.at[b]).wait()

    for b in range(NBUF):
        fire_gather(b, b)

    @pl.loop(0, BPW // NBUF)
    def _outer(o):
        for b in range(NBUF):
            i = o * NBUF + b
            sb = b % SNB
            wait_gather(b)
            if b >= SNB:
                pltpu.make_async_copy(
                    sbuf.at[sb], out_hbm.at[0], ssem.at[sb]).wait()
            else:
                @pl.when(o > 0)
                def _():
                    pltpu.make_async_copy(
                        sbuf.at[sb], out_hbm.at[0], ssem.at[sb]).wait()

            @pl.loop(0, MAX_LEN, unroll=4)
            def _row(r):
                for j in range(NVPR):
                    sl = pl.ds(j * 16, 16)
                    sbuf[sb, r, sl] = gbuf[b, r, sl] + pe_v[r, sl]

            @pl.when(o < BPW // NBUF - 1)
            def _():
                fire_gather(i + NBUF, b)

            pltpu.async_copy(sbuf.at[sb], out_hbm.at[wid * BPW + i], ssem.at[sb])

    for sb in range(SNB):
        pltpu.make_async_copy(sbuf.at[sb], out_hbm.at[0], ssem.at[sb]).wait()


def _positional_encoding() -> np.ndarray:
    pos = np.arange(MAX_LEN, dtype=np.float32)[:, None]
    i = np.arange(D_MODEL // 2, dtype=np.float32)[None, :]
    denom = np.power(10000.0, (2.0 * i) / D_MODEL)
    pe = np.zeros((MAX_LEN, D_MODEL), dtype=np.float32)
    pe[:, 0::2] = np.sin(pos / denom)
    pe[:, 1::2] = np.cos(pos / denom)
    return pe


def kernel(tokens, table):
    pe = jnp.asarray(_positional_encoding())
    tok3 = tokens.reshape(NW, 2 * BPW, G)
    return _sc_embed(tok3, table, pe)
